# baseline (speedup 1.0000x reference)
import functools
import math

import jax
import jax.numpy as jnp
from jax.experimental import pallas as pl
from jax.experimental.pallas import tpu as pltpu

_T = 2048
_C = 768
_H = 12
_HD = _C // _H
_RB = 1024
_GR = 1024
_N_ITER = 10


def _qkv_body(x_ref, w_ref, b_ref, o_ref):
    o_ref[...] = jax.lax.dot_general(
        x_ref[...], w_ref[...], (((1,), (1,)), ((), ())),
        preferred_element_type=jnp.float32) + b_ref[...]


def _attn_body(ratio_ref, q_ref, k_ref, v_ref, o_ref):
    hp = pl.program_id(0)
    g = pl.program_id(1)
    tb = pl.program_id(2)
    for g_st in range(_T // _GR):

        @pl.when(g == g_st)
        def _():
            _attn_group(g_st * _GR + _GR, g_st * _GR, hp, tb,
                        ratio_ref, q_ref, k_ref, v_ref, o_ref)


def _attn_group(W, ROFF, hp, tb, ratio_ref, q_ref, k_ref, v_ref, o_ref):
    scale = 1.0 / math.sqrt(_HD)
    rows = ROFF + tb * _RB + jax.lax.broadcasted_iota(jnp.int32, (_RB, 1), 0)
    cols = jax.lax.broadcasted_iota(jnp.int32, (_RB, W), 1)
    valid = cols <= rows
    big = jnp.float32(3e38)
    tlen = (rows + 1).astype(jnp.float32)

    att_m, mrow, lo, hi, ktf = [None, None], [None, None], [None, None], \
        [None, None], [None, None]
    for sub in range(2):
        q = q_ref[:, sub * _HD:(sub + 1) * _HD]
        k = k_ref[:W, sub * _HD:(sub + 1) * _HD]
        att = jax.lax.dot_general(
            q, k, (((1,), (1,)), ((), ())),
            preferred_element_type=jnp.float32) * scale
        att_m[sub] = jnp.where(valid, att, -big)
        mrow[sub] = jnp.max(att_m[sub], axis=1, keepdims=True)
        lo[sub] = jnp.min(jnp.where(valid, att, big), axis=1, keepdims=True)
        hi[sub] = mrow[sub]
        r = ratio_ref[2 * hp + sub]
        sig = 1.0 / (1.0 + jnp.exp(-r))
        kt = jnp.maximum(1, jnp.floor(tlen * sig).astype(jnp.int32))
        ktf[sub] = kt.astype(jnp.float32)

    for _ in range(_N_ITER):
        for sub in range(2):
            mid = (lo[sub] + hi[sub]) * 0.5
            cnt = jnp.sum((att_m[sub] >= mid).astype(jnp.float32), axis=1,
                          keepdims=True)
            ge = cnt >= ktf[sub]
            lo[sub] = jnp.where(ge, mid, lo[sub])
            hi[sub] = jnp.where(ge, hi[sub], mid)

    for sub in range(2):
        m = jnp.maximum(mrow[sub], 0.0)
        s = jnp.where(att_m[sub] >= lo[sub], att_m[sub], 0.0)
        p = jnp.exp(s - m)
        num = jax.lax.dot_general(
            p, v_ref[:W, sub * _HD:(sub + 1) * _HD], (((1,), (0,)), ((), ())),
            preferred_element_type=jnp.float32)
        den = jnp.sum(p, axis=1, keepdims=True)
        if W < _T:
            em = jnp.exp(-m)
            vtail = jnp.sum(v_ref[W:, sub * _HD:(sub + 1) * _HD], axis=0,
                            keepdims=True)
            num = num + em * vtail
            den = den + em * jnp.float32(_T - W)
        o_ref[:, sub * _HD:(sub + 1) * _HD] = num / den


def _proj_body(y_ref, w_ref, b_ref, o_ref):
    o_ref[...] = jax.lax.dot_general(
        y_ref[...], w_ref[...], (((1,), (1,)), ((), ())),
        preferred_element_type=jnp.float32) + b_ref[...]


@jax.jit
def kernel(x, W_attn, b_attn, W_proj, b_proj, sparsity_ratios):
    B, T, C = x.shape
    H = sparsity_ratios.shape[0]
    hd = C // H
    x2 = x.reshape(T, C)

    qkv = pl.pallas_call(
        _qkv_body,
        grid=(T // _RB,),
        in_specs=[
            pl.BlockSpec((_RB, C), lambda i: (i, 0)),
            pl.BlockSpec((3 * C, C), lambda i: (0, 0)),
            pl.BlockSpec((1, 3 * C), lambda i: (0, 0)),
        ],
        out_specs=pl.BlockSpec((_RB, 3 * C), lambda i: (i, 0)),
        out_shape=jax.ShapeDtypeStruct((T, 3 * C), jnp.float32),
    )(x2, W_attn, b_attn.reshape(1, 3 * C))

    hpairs = H // 2
    rpg = _GR // _RB
    grid_spec = pltpu.PrefetchScalarGridSpec(
        num_scalar_prefetch=1,
        grid=(hpairs, T // _GR, rpg),
        in_specs=[
            pl.BlockSpec((_RB, 128),
                         lambda h, g, t, *_: (g * rpg + t, h)),
            pl.BlockSpec((T, 128), lambda h, g, t, *_: (0, hpairs + h)),
            pl.BlockSpec((T, 128), lambda h, g, t, *_: (0, 2 * hpairs + h)),
        ],
        out_specs=pl.BlockSpec((_RB, 128),
                               lambda h, g, t, *_: (g * rpg + t, h)),
    )
    y2 = pl.pallas_call(
        _attn_body,
        grid_spec=grid_spec,
        out_shape=jax.ShapeDtypeStruct((T, C), jnp.float32),
        compiler_params=pltpu.CompilerParams(
            dimension_semantics=("arbitrary", "arbitrary", "arbitrary")),
    )(sparsity_ratios, qkv, qkv, qkv)
    out = pl.pallas_call(
        _proj_body,
        grid=(T // _RB,),
        in_specs=[
            pl.BlockSpec((_RB, C), lambda i: (i, 0)),
            pl.BlockSpec((C, C), lambda i: (0, 0)),
            pl.BlockSpec((1, C), lambda i: (0, 0)),
        ],
        out_specs=pl.BlockSpec((_RB, C), lambda i: (i, 0)),
        out_shape=jax.ShapeDtypeStruct((T, C), jnp.float32),
    )(y2, W_proj, b_proj.reshape(1, C))
    return out.reshape(B, T, C)


# speedup vs baseline: 1.0388x; 1.0388x over previous
import functools
import math

import jax
import jax.numpy as jnp
from jax.experimental import pallas as pl
from jax.experimental.pallas import tpu as pltpu

_T = 2048
_C = 768
_H = 12
_HD = _C // _H
_RB = 512
_GR = 512
_N_ITER = 10


def _qkv_body(x_ref, w_ref, b_ref, o_ref):
    o_ref[...] = jax.lax.dot_general(
        x_ref[...], w_ref[...], (((1,), (1,)), ((), ())),
        preferred_element_type=jnp.float32) + b_ref[...]


def _attn_body(ratio_ref, q_ref, k_ref, v_ref, o_ref):
    hp = pl.program_id(0)
    g = pl.program_id(1)
    tb = pl.program_id(2)
    for g_st in range(_T // _GR):

        @pl.when(g == g_st)
        def _():
            _attn_group(g_st * _GR + _GR, g_st * _GR, hp, tb,
                        ratio_ref, q_ref, k_ref, v_ref, o_ref)


def _attn_group(W, ROFF, hp, tb, ratio_ref, q_ref, k_ref, v_ref, o_ref):
    scale = 1.0 / math.sqrt(_HD)
    rows = ROFF + tb * _RB + jax.lax.broadcasted_iota(jnp.int32, (_RB, 1), 0)
    cols = jax.lax.broadcasted_iota(jnp.int32, (_RB, W), 1)
    valid = cols <= rows
    big = jnp.float32(3e38)
    tlen = (rows + 1).astype(jnp.float32)

    att_m, mrow, lo, hi, ktf = [None, None], [None, None], [None, None], \
        [None, None], [None, None]
    for sub in range(2):
        q = q_ref[:, sub * _HD:(sub + 1) * _HD]
        k = k_ref[:W, sub * _HD:(sub + 1) * _HD]
        att = jax.lax.dot_general(
            q, k, (((1,), (1,)), ((), ())),
            preferred_element_type=jnp.float32) * scale
        att_m[sub] = jnp.where(valid, att, -big)
        mrow[sub] = jnp.max(att_m[sub], axis=1, keepdims=True)
        lo[sub] = jnp.min(jnp.where(valid, att, big), axis=1, keepdims=True)
        hi[sub] = mrow[sub]
        r = ratio_ref[2 * hp + sub]
        sig = 1.0 / (1.0 + jnp.exp(-r))
        kt = jnp.maximum(1, jnp.floor(tlen * sig).astype(jnp.int32))
        ktf[sub] = kt.astype(jnp.float32)

    for _ in range(_N_ITER):
        for sub in range(2):
            mid = (lo[sub] + hi[sub]) * 0.5
            cnt = jnp.sum((att_m[sub] >= mid).astype(jnp.float32), axis=1,
                          keepdims=True)
            ge = cnt >= ktf[sub]
            lo[sub] = jnp.where(ge, mid, lo[sub])
            hi[sub] = jnp.where(ge, hi[sub], mid)

    for sub in range(2):
        m = jnp.maximum(mrow[sub], 0.0)
        s = jnp.where(att_m[sub] >= lo[sub], att_m[sub], 0.0)
        p = jnp.exp(s - m)
        num = jax.lax.dot_general(
            p, v_ref[:W, sub * _HD:(sub + 1) * _HD], (((1,), (0,)), ((), ())),
            preferred_element_type=jnp.float32)
        den = jnp.sum(p, axis=1, keepdims=True)
        if W < _T:
            em = jnp.exp(-m)
            vtail = jnp.sum(v_ref[W:, sub * _HD:(sub + 1) * _HD], axis=0,
                            keepdims=True)
            num = num + em * vtail
            den = den + em * jnp.float32(_T - W)
        o_ref[:, sub * _HD:(sub + 1) * _HD] = num / den


def _proj_body(y_ref, w_ref, b_ref, o_ref):
    o_ref[...] = jax.lax.dot_general(
        y_ref[...], w_ref[...], (((1,), (1,)), ((), ())),
        preferred_element_type=jnp.float32) + b_ref[...]


@jax.jit
def kernel(x, W_attn, b_attn, W_proj, b_proj, sparsity_ratios):
    B, T, C = x.shape
    H = sparsity_ratios.shape[0]
    hd = C // H
    x2 = x.reshape(T, C)

    qkv = pl.pallas_call(
        _qkv_body,
        grid=(T // _RB,),
        in_specs=[
            pl.BlockSpec((_RB, C), lambda i: (i, 0)),
            pl.BlockSpec((3 * C, C), lambda i: (0, 0)),
            pl.BlockSpec((1, 3 * C), lambda i: (0, 0)),
        ],
        out_specs=pl.BlockSpec((_RB, 3 * C), lambda i: (i, 0)),
        out_shape=jax.ShapeDtypeStruct((T, 3 * C), jnp.float32),
    )(x2, W_attn, b_attn.reshape(1, 3 * C))

    hpairs = H // 2
    rpg = _GR // _RB
    grid_spec = pltpu.PrefetchScalarGridSpec(
        num_scalar_prefetch=1,
        grid=(hpairs, T // _GR, rpg),
        in_specs=[
            pl.BlockSpec((_RB, 128),
                         lambda h, g, t, *_: (g * rpg + t, h)),
            pl.BlockSpec((T, 128), lambda h, g, t, *_: (0, hpairs + h)),
            pl.BlockSpec((T, 128), lambda h, g, t, *_: (0, 2 * hpairs + h)),
        ],
        out_specs=pl.BlockSpec((_RB, 128),
                               lambda h, g, t, *_: (g * rpg + t, h)),
    )
    y2 = pl.pallas_call(
        _attn_body,
        grid_spec=grid_spec,
        out_shape=jax.ShapeDtypeStruct((T, C), jnp.float32),
        compiler_params=pltpu.CompilerParams(
            dimension_semantics=("arbitrary", "arbitrary", "arbitrary")),
    )(sparsity_ratios, qkv, qkv, qkv)
    out = pl.pallas_call(
        _proj_body,
        grid=(T // _RB,),
        in_specs=[
            pl.BlockSpec((_RB, C), lambda i: (i, 0)),
            pl.BlockSpec((C, C), lambda i: (0, 0)),
            pl.BlockSpec((1, C), lambda i: (0, 0)),
        ],
        out_specs=pl.BlockSpec((_RB, C), lambda i: (i, 0)),
        out_shape=jax.ShapeDtypeStruct((T, C), jnp.float32),
    )(y2, W_proj, b_proj.reshape(1, C))
    return out.reshape(B, T, C)
